# baseline (speedup 1.0000x reference)
import jax
import jax.numpy as jnp
from jax import lax
from jax.experimental import pallas as pl
from jax.experimental.pallas import tpu as pltpu
from jax.experimental.pallas import tpu_sc as plsc

_NUM_CORES = 2
_NUM_SUBCORES = 16
_NW = _NUM_CORES * _NUM_SUBCORES

_SEQ = 2048
_VOCAB = 50257
_TILE = 128
_V_MAIN = (_VOCAB // (28 * _TILE)) * (28 * _TILE)
_CW = 3584
_NCHUNK = _V_MAIN // _CW
_TV = 2048
_HALF = _SEQ // 2


def _gather_body(tok_hbm, idx_hbm, out_hbm, idx_v, rows_v, sem):
    b_per_w = idx_v.shape[0]
    wid = lax.axis_index("s") * _NUM_CORES + lax.axis_index("c")
    base = wid * b_per_w
    pltpu.sync_copy(idx_hbm.at[pl.ds(base, b_per_w)], idx_v)
    pltpu.async_copy(tok_hbm.at[idx_v], rows_v, sem).wait()
    pltpu.sync_copy(rows_v, out_hbm.at[pl.ds(base, b_per_w)])


def _sc_gather(tok_emb, idx):
    seq = idx.shape[0]
    emb = tok_emb.shape[1]
    b_per_w = seq // _NW
    mesh = plsc.VectorSubcoreMesh(core_axis_name="c", subcore_axis_name="s")
    return pl.kernel(
        _gather_body,
        mesh=mesh,
        out_type=jax.ShapeDtypeStruct((seq, emb), jnp.float32),
        scratch_types=[
            pltpu.VMEM((b_per_w,), jnp.int32),
            pltpu.VMEM((b_per_w, emb), jnp.float32),
            pltpu.SemaphoreType.DMA,
        ],
    )(tok_emb, idx)


def _mmpack_body(x_ref, pos_ref, w_ref, packed_ref, tail_ref):
    v = pl.program_id(0)
    x = x_ref[...] + pos_ref[...]
    y = lax.dot_general(
        x, w_ref[...], (((1,), (1,)), ((), ())),
        preferred_element_type=jnp.float32)
    lo = lax.bitcast_convert_type(y[:_HALF], jnp.uint32)
    hi = lax.bitcast_convert_type(y[_HALF:], jnp.uint32)
    packed_ref[...] = lax.bitcast_convert_type(
        (lo >> 16) | (hi & jnp.uint32(0xFFFF0000)), jnp.float32)
    @pl.when(v == pl.num_programs(0) - 1)
    def _():
        off = _V_MAIN - (pl.cdiv(_V_MAIN + _TILE, _TV) - 1) * _TV
        tail_ref[...] = y[:, off:off + _TILE]


def _tc_mmpack(x, pos_emb, w_out):
    seq, emb = x.shape
    grid = (24,)
    return pl.pallas_call(
        _mmpack_body,
        grid=grid,
        in_specs=[
            pl.BlockSpec((seq, emb), lambda v: (0, 0)),
            pl.BlockSpec((seq, emb), lambda v: (0, 0)),
            pl.BlockSpec((_TV, emb), lambda v: (v, 0)),
        ],
        out_specs=[
            pl.BlockSpec((_HALF, _TV), lambda v: (0, v)),
            pl.BlockSpec((seq, _TILE), lambda v: (0, 0)),
        ],
        out_shape=[
            jax.ShapeDtypeStruct((_HALF, _VOCAB), jnp.float32),
            jax.ShapeDtypeStruct((seq, _TILE), jnp.float32),
        ],
    )(x, pos_emb, w_out)


def _upcast_body(packed_hbm, out_hbm,
                 bin0, bin1, bhi0, bhi1,
                 si0, si1, slo0, slo1, shi0, shi1):
    wid = lax.axis_index("s") * _NUM_CORES + lax.axis_index("c")
    bins = (bin0, bin1)
    bhis = (bhi0, bhi1)
    sis = (si0, si1)
    slos = (slo0, slo1)
    shis = (shi0, shi1)

    def start_in(j, k, buf, sem):
        rows = wid * 32 + j * 8
        return pltpu.async_copy(
            packed_hbm.at[pl.ds(rows, 8), pl.ds(k * _CW, _CW)], buf, sem)

    def start_outs(j, k, blo, bhi, semlo, semhi):
        rows = wid * 32 + j * 8
        h1 = pltpu.async_copy(
            blo, out_hbm.at[0, pl.ds(rows, 8), pl.ds(k * _CW, _CW)], semlo)
        h2 = pltpu.async_copy(
            bhi, out_hbm.at[0, pl.ds(_HALF + rows, 8), pl.ds(k * _CW, _CW)],
            semhi)
        return h1, h2

    def compute(bin_, bhi):
        mask = jnp.uint32(0xFFFF0000)
        def step(i, c):
            for p in range(8):
                w = lax.bitcast_convert_type(bin_[p, pl.ds(i * 16, 16)],
                                             jnp.uint32)
                bhi[p, pl.ds(i * 16, 16)] = lax.bitcast_convert_type(
                    w & mask, jnp.float32)
                bin_[p, pl.ds(i * 16, 16)] = lax.bitcast_convert_type(
                    w << 16, jnp.float32)
            return c
        lax.fori_loop(0, _CW // 16, step, 0)

    steps = [(j, k) for j in range(4) for k in range(_NCHUNK)]
    in_h = [None, None]
    out_h = [None, None]
    in_h[0] = start_in(steps[0][0], steps[0][1], bins[0], sis[0])
    for idx, (j, k) in enumerate(steps):
        p = idx % 2
        in_h[p].wait()
        if idx + 1 < len(steps):
            q = (idx + 1) % 2
            jn, kn = steps[idx + 1]
            if out_h[q] is not None:
                out_h[q][0].wait()
                out_h[q][1].wait()
                out_h[q] = None
            in_h[q] = start_in(jn, kn, bins[q], sis[q])
        compute(bins[p], bhis[p])
        out_h[p] = start_outs(j, k, bins[p], bhis[p], slos[p], shis[p])
    for p in (0, 1):
        if out_h[p] is not None:
            out_h[p][0].wait()
            out_h[p][1].wait()


def _sc_upcast(packed):
    mesh = plsc.VectorSubcoreMesh(core_axis_name="c", subcore_axis_name="s")
    return pl.kernel(
        _upcast_body,
        mesh=mesh,
        out_type=jax.ShapeDtypeStruct((1, _SEQ, _VOCAB), jnp.float32),
        scratch_types=[
            pltpu.VMEM((8, _CW), jnp.float32),
            pltpu.VMEM((8, _CW), jnp.float32),
            pltpu.VMEM((8, _CW), jnp.float32),
            pltpu.VMEM((8, _CW), jnp.float32),
            pltpu.SemaphoreType.DMA,
            pltpu.SemaphoreType.DMA,
            pltpu.SemaphoreType.DMA,
            pltpu.SemaphoreType.DMA,
            pltpu.SemaphoreType.DMA,
            pltpu.SemaphoreType.DMA,
        ],
    )(packed)


def _tailfix_body(big_ref, tail_ref, out_ref):
    out_ref[...] = tail_ref[...]


def _tc_tailfix(big, tail):
    return pl.pallas_call(
        _tailfix_body,
        grid=(1,),
        in_specs=[
            pl.BlockSpec(memory_space=pl.ANY),
            pl.BlockSpec((_SEQ, _TILE), lambda i: (0, 0)),
        ],
        out_specs=pl.BlockSpec((None, _SEQ, _TILE),
                               lambda i: (0, 0, _V_MAIN // _TILE)),
        out_shape=jax.ShapeDtypeStruct((1, _SEQ, _VOCAB), jnp.float32),
        input_output_aliases={0: 0},
    )(big, tail)


def _mm_body(x_ref, pos_ref, w_ref, out_ref):
    x = x_ref[...] + pos_ref[...]
    out_ref[...] = lax.dot_general(
        x, w_ref[...], (((1,), (1,)), ((), ())),
        preferred_element_type=jnp.float32)


def _tc_matmul_rows(x, pos_emb, w_out, row0, nrows):
    seq, emb = x.shape
    vocab = w_out.shape[0]
    ntiles = pl.cdiv(vocab, _TV)
    return pl.pallas_call(
        _mm_body,
        grid=(ntiles,),
        in_specs=[
            pl.BlockSpec((nrows, emb), lambda v: (row0, 0)),
            pl.BlockSpec((nrows, emb), lambda v: (row0, 0)),
            pl.BlockSpec((_TV, emb), lambda v: (v, 0)),
        ],
        out_specs=pl.BlockSpec((nrows, _TV), lambda v: (0, v)),
        out_shape=jax.ShapeDtypeStruct((nrows, vocab), jnp.float32),
    )(x, pos_emb, w_out)


def kernel(in_idx, tok_emb, pos_emb, W_out):
    batch, seq = in_idx.shape
    vocab = W_out.shape[0]
    idx = in_idx.reshape(seq).astype(jnp.int32)
    x = _sc_gather(tok_emb, idx)
    n_chunks = 4
    nrows = seq // n_chunks
    chunks = []
    for k in range(n_chunks):
        y = _tc_matmul_rows(x, pos_emb, W_out, k, nrows)
        chunks.append(y.reshape(1, nrows, vocab))
    return jnp.concatenate(chunks, axis=1)


# speedup vs baseline: 1.7244x; 1.7244x over previous
import jax
import jax.numpy as jnp
from jax import lax
from jax.experimental import pallas as pl
from jax.experimental.pallas import tpu as pltpu
from jax.experimental.pallas import tpu_sc as plsc

_NUM_CORES = 2
_NUM_SUBCORES = 16
_NW = _NUM_CORES * _NUM_SUBCORES
_TV = 2048


def _gather_body(tok_hbm, idx_hbm, out_hbm, idx_v, rows_v, sem):
    b_per_w = idx_v.shape[0]
    wid = lax.axis_index("s") * _NUM_CORES + lax.axis_index("c")
    base = wid * b_per_w
    pltpu.sync_copy(idx_hbm.at[pl.ds(base, b_per_w)], idx_v)
    pltpu.async_copy(tok_hbm.at[idx_v], rows_v, sem).wait()
    pltpu.sync_copy(rows_v, out_hbm.at[pl.ds(base, b_per_w)])


def _sc_gather(tok_emb, idx):
    seq = idx.shape[0]
    emb = tok_emb.shape[1]
    b_per_w = seq // _NW
    mesh = plsc.VectorSubcoreMesh(core_axis_name="c", subcore_axis_name="s")
    return pl.kernel(
        _gather_body,
        mesh=mesh,
        out_type=jax.ShapeDtypeStruct((seq, emb), jnp.float32),
        scratch_types=[
            pltpu.VMEM((b_per_w,), jnp.int32),
            pltpu.VMEM((b_per_w, emb), jnp.float32),
            pltpu.SemaphoreType.DMA,
        ],
    )(tok_emb, idx)


def _matmul_body(x_ref, pos_ref, w_ref, out_ref):
    x = x_ref[...] + pos_ref[...]
    out_ref[...] = lax.dot_general(
        x, w_ref[...], (((1,), (1,)), ((), ())),
        preferred_element_type=jnp.float32)


def _tc_matmul(x, pos_emb, w_out):
    seq, emb = x.shape
    vocab = w_out.shape[0]
    grid = (pl.cdiv(vocab, _TV),)
    return pl.pallas_call(
        _matmul_body,
        grid=grid,
        in_specs=[
            pl.BlockSpec((seq, emb), lambda v: (0, 0)),
            pl.BlockSpec((seq, emb), lambda v: (0, 0)),
            pl.BlockSpec((_TV, emb), lambda v: (v, 0)),
        ],
        out_specs=pl.BlockSpec((seq, _TV), lambda v: (0, v)),
        out_shape=jax.ShapeDtypeStruct((seq, vocab), jnp.float32),
    )(x, pos_emb, w_out)


def kernel(in_idx, tok_emb, pos_emb, W_out):
    batch, seq = in_idx.shape
    vocab, emb = W_out.shape
    idx = in_idx.reshape(seq).astype(jnp.int32)
    x = _sc_gather(tok_emb, idx)
    logits = _tc_matmul(x, pos_emb, W_out)
    return logits.reshape(batch, seq, vocab)
